# baseline (speedup 1.0000x reference)
import jax
import jax.numpy as jnp
from jax.experimental import pallas as pl
from jax.experimental.pallas import tpu as pltpu

_N = 10000
_NH = 128

_IB = 200
_NI = _N // _IB
_NBUF = 4
_HALF = _N // 2


def _agg_body(adj_ref, x_ref, w_ref, b_ref, o_ref,
              buf_ref, xv_ref, stg_ref, acc_ref, deg_ref,
              sem, xsem, osem):
    def start_copy(k, slot):
        pltpu.make_async_copy(
            adj_ref.at[pl.ds(k * _IB, _IB), :],
            buf_ref.at[slot],
            sem.at[slot],
        ).start()

    xcopy0 = pltpu.make_async_copy(
        x_ref.at[pl.ds(0, _IB), :], xv_ref.at[pl.ds(0, _IB), :],
        xsem.at[0])
    xcopy0.start()
    for k in range(_NBUF):
        start_copy(k, k)
    xcopy1 = pltpu.make_async_copy(
        x_ref.at[pl.ds(_IB, _N - _IB), :], xv_ref.at[pl.ds(_IB, _N - _IB), :],
        xsem.at[1])
    xcopy1.start()
    xcopy0.wait()

    for k in range(_NI):
        slot = k % _NBUF
        pltpu.make_async_copy(
            adj_ref.at[pl.ds(k * _IB, _IB), :],
            buf_ref.at[slot],
            sem.at[slot],
        ).wait()
        if k == 1:
            xcopy1.wait()
        sup = jnp.maximum(
            jax.lax.dot_general(
                xv_ref[k * _IB:(k + 1) * _IB, :], w_ref[...],
                (((1,), (1,)), ((), ())),
                preferred_element_type=jnp.float32) + b_ref[...],
            0.0).astype(jnp.bfloat16)
        sel = jnp.where(buf_ref[slot] > 0.0, 1.0, 0.0)
        dsum = jnp.sum(sel, axis=0, keepdims=True)
        mask = sel.astype(jnp.bfloat16)
        part = jax.lax.dot_general(
            sup, mask, (((0,), (0,)), ((), ())),
            preferred_element_type=jnp.float32)
        if k == 0:
            deg_ref[...] = dsum
            acc_ref[...] = part
        else:
            deg_ref[...] += dsum
            acc_ref[...] += part
        if k + _NBUF < _NI:
            start_copy(k + _NBUF, slot)

    ocopies = []
    for h in range(2):
        cols = slice(h * _HALF, (h + 1) * _HALF)
        rows = pl.ds(h * _HALF, _HALF)
        stg_ref[rows, :] = jnp.transpose(
            acc_ref[:, cols] / deg_ref[:, cols])
        oc = pltpu.make_async_copy(
            stg_ref.at[rows, :], o_ref.at[rows, :], osem.at[h])
        oc.start()
        ocopies.append(oc)
    for oc in ocopies:
        oc.wait()


def kernel(input, adj, W, b):
    return pl.pallas_call(
        _agg_body,
        in_specs=[
            pl.BlockSpec(memory_space=pl.ANY),
            pl.BlockSpec(memory_space=pl.ANY),
            pl.BlockSpec(memory_space=pltpu.MemorySpace.VMEM),
            pl.BlockSpec(memory_space=pltpu.MemorySpace.VMEM),
        ],
        out_specs=pl.BlockSpec(memory_space=pl.ANY),
        out_shape=jax.ShapeDtypeStruct((_N, _NH), jnp.float32),
        scratch_shapes=[
            pltpu.VMEM((_NBUF, _IB, _N), jnp.float32),
            pltpu.VMEM((_N, _NH), jnp.float32),
            pltpu.VMEM((_N, _NH), jnp.float32),
            pltpu.VMEM((_NH, _N), jnp.float32),
            pltpu.VMEM((1, _N), jnp.float32),
            pltpu.SemaphoreType.DMA((_NBUF,)),
            pltpu.SemaphoreType.DMA((2,)),
            pltpu.SemaphoreType.DMA((2,)),
        ],
    )(adj, input, W, b.reshape(1, _NH))
